# baseline (speedup 1.0000x reference)
import functools

import jax
import jax.numpy as jnp
from jax import lax
from jax.experimental import pallas as pl
from jax.experimental.pallas import tpu as pltpu
from jax.experimental.pallas import tpu_sc as plsc

N = 50000
E = 800000
S = 9
CO = 16
BATCH = 100
LIN_IN = 500 * 16
LIN_OUT = 8

NC = 2
NS = 16
NW = NC * NS

NA = 51200
EP = 819200
EPW = EP // NW
CH = 1024
NCHUNK = EPW // CH
RPC = CH // 128
ROWS_PW = EPW // 128
NPS = NA // NS

BE = 2048
BN = 2048

_mesh = functools.partial(
    plsc.VectorSubcoreMesh, core_axis_name="c", subcore_axis_name="s")
_sc_params = pltpu.CompilerParams(use_tc_tiling_on_sc=False)


@functools.lru_cache(maxsize=None)
def _make_gather(nrows, ci):
  @functools.partial(
      pl.kernel,
      out_type=jax.ShapeDtypeStruct((EP, ci), jnp.float32),
      mesh=_mesh(),
      scratch_types=[
          pltpu.VMEM((RPC, 128), jnp.int32),
          pltpu.VMEM((CH, ci), jnp.float32),
          pltpu.SemaphoreType.DMA,
      ],
      compiler_params=_sc_params,
  )
  def gather_k(table, src2, out, idx_v, rows_v, sem):
    cid = lax.axis_index("c")
    sid = lax.axis_index("s")
    wid = sid * NC + cid
    ebase = wid * EPW
    rbase = wid * ROWS_PW

    def step(i, carry):
      pltpu.sync_copy(src2.at[pl.ds(rbase + i * RPC, RPC)], idx_v)
      cps = [
          pltpu.async_copy(table.at[idx_v.at[j]],
                           rows_v.at[pl.ds(j * 128, 128)], sem)
          for j in range(RPC)
      ]
      for cp in cps:
        cp.wait()
      pltpu.sync_copy(rows_v, out.at[pl.ds(ebase + i * CH, CH)])
      return carry

    lax.fori_loop(0, NCHUNK, step, 0)

  return gather_k


def _scatter_body(msg, dst2, zeros, out, idx_v, msg_v, acc):
  cid = lax.axis_index("c")
  sid = lax.axis_index("s")
  wid = sid * NC + cid
  ebase = wid * EPW
  rbase = wid * ROWS_PW

  pltpu.sync_copy(zeros.at[pl.ds(sid * NPS, NPS)],
                  acc.at[pl.ds(sid * NPS, NPS)])
  plsc.subcore_barrier()

  def step(i, carry):
    pltpu.sync_copy(dst2.at[pl.ds(rbase + i * RPC, RPC)], idx_v)
    pltpu.sync_copy(msg.at[pl.ds(ebase + i * CH, CH)], msg_v)
    for j in range(RPC):
      pltpu.sync_copy(msg_v.at[pl.ds(j * 128, 128)],
                      acc.at[idx_v.at[j]], add=True)
    return carry

  lax.fori_loop(0, NCHUNK, step, 0)
  plsc.subcore_barrier()
  pltpu.sync_copy(acc.at[pl.ds(sid * NPS, NPS)],
                  out.at[pl.ds(cid * NA + sid * NPS, NPS)])


@functools.partial(
    pl.kernel,
    out_type=jax.ShapeDtypeStruct((2 * NA, CO), jnp.float32),
    mesh=_mesh(),
    scratch_types=[
        pltpu.VMEM((RPC, 128), jnp.int32),
        pltpu.VMEM((CH, CO), jnp.float32),
        pltpu.VMEM_SHARED((NA, CO), jnp.float32),
    ],
    compiler_params=_sc_params,
)
def _scatter_k(msg, dst2, zeros, out, idx_v, msg_v, acc):
  _scatter_body(msg, dst2, zeros, out, idx_v, msg_v, acc)


@functools.partial(
    pl.kernel,
    out_type=jax.ShapeDtypeStruct((4 * NA, CO), jnp.float32),
    mesh=_mesh(),
    scratch_types=[
        pltpu.VMEM((RPC, 128), jnp.int32),
        pltpu.VMEM((CH, CO), jnp.float32),
        pltpu.VMEM((128, CO), jnp.float32),
        pltpu.VMEM_SHARED((NA, CO), jnp.float32),
        pltpu.VMEM_SHARED((NA, CO), jnp.float32),
    ],
    compiler_params=_sc_params,
)
def _scatter_deg_k(msg, dst2, zeros, ones, out, idx_v, msg_v, ones_v,
                   acc, acc_d):
  cid = lax.axis_index("c")
  sid = lax.axis_index("s")
  wid = sid * NC + cid
  ebase = wid * EPW
  rbase = wid * ROWS_PW

  pltpu.sync_copy(zeros.at[pl.ds(sid * NPS, NPS)],
                  acc.at[pl.ds(sid * NPS, NPS)])
  pltpu.sync_copy(zeros.at[pl.ds(sid * NPS, NPS)],
                  acc_d.at[pl.ds(sid * NPS, NPS)])
  pltpu.sync_copy(ones, ones_v)
  plsc.subcore_barrier()

  def step(i, carry):
    pltpu.sync_copy(dst2.at[pl.ds(rbase + i * RPC, RPC)], idx_v)
    pltpu.sync_copy(msg.at[pl.ds(ebase + i * CH, CH)], msg_v)
    for j in range(RPC):
      pltpu.sync_copy(msg_v.at[pl.ds(j * 128, 128)],
                      acc.at[idx_v.at[j]], add=True)
      pltpu.sync_copy(ones_v, acc_d.at[idx_v.at[j]], add=True)
    return carry

  lax.fori_loop(0, NCHUNK, step, 0)
  plsc.subcore_barrier()
  pltpu.sync_copy(acc.at[pl.ds(sid * NPS, NPS)],
                  out.at[pl.ds(cid * NA + sid * NPS, NPS)])
  pltpu.sync_copy(acc_d.at[pl.ds(sid * NPS, NPS)],
                  out.at[pl.ds((2 + cid) * NA + sid * NPS, NPS)])


def _msg_body(pos_ref, xj_ref, w_ref, out_ref):
  pos = pos_ref[...]
  t = pos - jnp.floor(pos)
  t0 = t[:, 0:1]
  t1 = t[:, 1:2]

  def fs(tt):
    return (0.5 * tt * tt - tt + 0.5, -tt * tt + tt + 0.5, 0.5 * tt * tt)

  f0 = fs(t0)
  f1 = fs(t1)
  xj = xj_ref[...]
  pieces = []
  for a in range(3):
    for b in range(3):
      pieces.append((f1[a] * f0[b]) * xj)
  amat = jnp.concatenate(pieces, axis=1)
  out_ref[...] = jnp.dot(amat, w_ref[...],
                         preferred_element_type=jnp.float32)


@functools.lru_cache(maxsize=None)
def _make_msg(ci):
  return pl.pallas_call(
      _msg_body,
      grid=(EP // BE,),
      in_specs=[
          pl.BlockSpec((BE, 2), lambda i: (i, 0)),
          pl.BlockSpec((BE, ci), lambda i: (i, 0)),
          pl.BlockSpec((S * ci, CO), lambda i: (0, 0)),
      ],
      out_specs=pl.BlockSpec((BE, CO), lambda i: (i, 0)),
      out_shape=jax.ShapeDtypeStruct((EP, CO), jnp.float32),
  )


def _dinv_body(d0_ref, d1_ref, out_ref):
  deg = d0_ref[...] + d1_ref[...]
  out_ref[...] = 1.0 / jnp.maximum(deg, 1.0)


_dinv_k = pl.pallas_call(
    _dinv_body,
    grid=(NA // BN,),
    in_specs=[
        pl.BlockSpec((BN, CO), lambda i: (i + 2 * (NA // BN), 0)),
        pl.BlockSpec((BN, CO), lambda i: (i + 3 * (NA // BN), 0)),
    ],
    out_specs=pl.BlockSpec((BN, CO), lambda i: (i, 0)),
    out_shape=jax.ShapeDtypeStruct((NA, CO), jnp.float32),
)


def _comb_body(p0_ref, p1_ref, dinv_ref, x_ref, root_ref, b_ref, out_ref):
  agg = (p0_ref[...] + p1_ref[...]) * dinv_ref[...]
  o = agg + jnp.dot(x_ref[...], root_ref[...],
                    preferred_element_type=jnp.float32) + b_ref[...]
  out_ref[...] = jnp.minimum(jnp.maximum(o, 0.0), 6.0)


@functools.lru_cache(maxsize=None)
def _make_combine(ci):
  return pl.pallas_call(
      _comb_body,
      grid=(NA // BN,),
      in_specs=[
          pl.BlockSpec((BN, CO), lambda i: (i, 0)),
          pl.BlockSpec((BN, CO), lambda i: (i + NA // BN, 0)),
          pl.BlockSpec((BN, CO), lambda i: (i, 0)),
          pl.BlockSpec((BN, ci), lambda i: (i, 0)),
          pl.BlockSpec((ci, CO), lambda i: (0, 0)),
          pl.BlockSpec((1, CO), lambda i: (0, 0)),
      ],
      out_specs=pl.BlockSpec((BN, CO), lambda i: (i, 0)),
      out_shape=jax.ShapeDtypeStruct((NA, CO), jnp.float32),
  )


def _comb_cat_body(p0_ref, p1_ref, dinv_ref, x_ref, root_ref, b_ref,
                   skip_ref, out_ref):
  agg = (p0_ref[...] + p1_ref[...]) * dinv_ref[...]
  o = agg + jnp.dot(x_ref[...], root_ref[...],
                    preferred_element_type=jnp.float32) + b_ref[...]
  o = jnp.minimum(jnp.maximum(o, 0.0), 6.0)
  out_ref[...] = jnp.concatenate([o, skip_ref[...]], axis=1)


@functools.lru_cache(maxsize=None)
def _make_combine_cat(ci):
  return pl.pallas_call(
      _comb_cat_body,
      grid=(NA // BN,),
      in_specs=[
          pl.BlockSpec((BN, CO), lambda i: (i, 0)),
          pl.BlockSpec((BN, CO), lambda i: (i + NA // BN, 0)),
          pl.BlockSpec((BN, CO), lambda i: (i, 0)),
          pl.BlockSpec((BN, ci), lambda i: (i, 0)),
          pl.BlockSpec((ci, CO), lambda i: (0, 0)),
          pl.BlockSpec((1, CO), lambda i: (0, 0)),
          pl.BlockSpec((BN, CO), lambda i: (i, 0)),
      ],
      out_specs=pl.BlockSpec((BN, 2 * CO), lambda i: (i, 0)),
      out_shape=jax.ShapeDtypeStruct((NA, 2 * CO), jnp.float32),
  )


def _final_body(flat_ref, w_ref, b_ref, batch_ref, out_ref):
  m = jnp.max(batch_ref[...])
  delta = (m + 1 - BATCH).astype(jnp.float32)
  out_ref[...] = jnp.dot(flat_ref[...], w_ref[...],
                         preferred_element_type=jnp.float32) \
      + b_ref[...] + delta


_final_k = pl.pallas_call(
    _final_body,
    out_shape=jax.ShapeDtypeStruct((BATCH, LIN_OUT), jnp.float32),
)


def kernel(x, edge_index, edge_attr, batch, pos, params):
  del edge_attr
  f32 = jnp.float32
  src = edge_index[0]
  dst = edge_index[1]
  pad_e = EP - E
  src2 = jnp.concatenate(
      [src, jnp.zeros((pad_e,), jnp.int32)]).reshape(EP // 128, 128)
  dst2 = jnp.concatenate(
      [dst, jnp.full((pad_e,), N, jnp.int32)]).reshape(EP // 128, 128)
  posp = jnp.concatenate([pos, jnp.zeros((pad_e, 2), f32)], axis=0)
  zeros_n = jnp.zeros((NA, CO), f32)
  ones_sc = jnp.ones((128, CO), f32)

  xpad = jnp.zeros((NA, CO), f32).at[:N].set(x)

  def wmat(l, ci):
    return params['conv%d_w' % l].reshape(S * ci, CO)

  def spmm(h, ci, l):
    xj = _make_gather(NA, ci)(h, src2)
    msg = _make_msg(ci)(posp, xj, wmat(l, ci))
    return _scatter_k(msg, dst2, zeros_n)

  def cargs(l):
    return params['conv%d_root' % l], params['conv%d_b' % l].reshape(1, CO)

  xj = _make_gather(NA, CO)(xpad, src2)
  msg = _make_msg(CO)(posp, xj, wmat(0, CO))
  p = _scatter_deg_k(msg, dst2, zeros_n, ones_sc)
  dinv = _dinv_k(p, p)
  o1 = _make_combine(CO)(p, p, dinv, xpad, *cargs(0))
  p = spmm(o1, CO, 1)
  o2 = _make_combine(CO)(p, p, dinv, o1, *cargs(1))
  p = spmm(o2, CO, 2)
  cat32 = _make_combine_cat(CO)(p, p, dinv, o2, *cargs(2), o2)
  p = spmm(cat32, 2 * CO, 3)
  cat32 = _make_combine_cat(2 * CO)(p, p, dinv, cat32, *cargs(3), o1)
  p = spmm(cat32, 2 * CO, 4)
  d = _make_combine(2 * CO)(p, p, dinv, cat32, *cargs(4))

  flat = d[:N].reshape(BATCH, LIN_IN)
  batch2 = batch.reshape(BATCH, N // BATCH)
  return _final_k(flat, params['lin_w'],
                  params['lin_b'].reshape(1, LIN_OUT), batch2)


# speedup vs baseline: 2.1531x; 2.1531x over previous
import functools

import jax
import jax.numpy as jnp
from jax import lax
from jax.experimental import pallas as pl
from jax.experimental.pallas import tpu as pltpu
from jax.experimental.pallas import tpu_sc as plsc

N = 50000
E = 800000
S = 9
CO = 16
BATCH = 100
LIN_IN = 500 * 16
LIN_OUT = 8

NC = 2
NS = 16
NW = NC * NS

NA = 51200
EP = 819200
EPW = EP // NW
CH = 1024
NCHUNK = EPW // CH
RPC = CH // 128
ROWS_PW = EPW // 128
NPS = NA // NS

BE = 2048
BN = 2048

_mesh = functools.partial(
    plsc.VectorSubcoreMesh, core_axis_name="c", subcore_axis_name="s")
_sc_params = pltpu.CompilerParams(use_tc_tiling_on_sc=False)


@functools.lru_cache(maxsize=None)
def _make_gather(nrows, ci):
  @functools.partial(
      pl.kernel,
      out_type=jax.ShapeDtypeStruct((EP, ci), jnp.float32),
      mesh=_mesh(),
      scratch_types=[
          pltpu.VMEM((RPC, 128), jnp.int32),
          pltpu.VMEM((CH, ci), jnp.float32),
          pltpu.SemaphoreType.DMA,
      ],
      compiler_params=_sc_params,
  )
  def gather_k(table, src2, out, idx_v, rows_v, sem):
    cid = lax.axis_index("c")
    sid = lax.axis_index("s")
    wid = sid * NC + cid
    ebase = wid * EPW
    rbase = wid * ROWS_PW

    def step(i, carry):
      pltpu.sync_copy(src2.at[pl.ds(rbase + i * RPC, RPC)], idx_v)
      cps = [
          pltpu.async_copy(table.at[idx_v.at[j]],
                           rows_v.at[pl.ds(j * 128, 128)], sem)
          for j in range(RPC)
      ]
      for cp in cps:
        cp.wait()
      pltpu.sync_copy(rows_v, out.at[pl.ds(ebase + i * CH, CH)])
      return carry

    lax.fori_loop(0, NCHUNK, step, 0)

  return gather_k


def _scatter_body(msg, dst2, zeros, out, idx_v, msg_v, acc):
  cid = lax.axis_index("c")
  sid = lax.axis_index("s")
  wid = sid * NC + cid
  ebase = wid * EPW
  rbase = wid * ROWS_PW

  pltpu.sync_copy(zeros.at[pl.ds(sid * NPS, NPS)],
                  acc.at[pl.ds(sid * NPS, NPS)])
  plsc.subcore_barrier()

  def step(i, carry):
    pltpu.sync_copy(dst2.at[pl.ds(rbase + i * RPC, RPC)], idx_v)
    pltpu.sync_copy(msg.at[pl.ds(ebase + i * CH, CH)], msg_v)
    for j in range(RPC):
      pltpu.sync_copy(msg_v.at[pl.ds(j * 128, 128)],
                      acc.at[idx_v.at[j]], add=True)
    return carry

  lax.fori_loop(0, NCHUNK, step, 0)
  plsc.subcore_barrier()
  pltpu.sync_copy(acc.at[pl.ds(sid * NPS, NPS)],
                  out.at[pl.ds(cid * NA + sid * NPS, NPS)])


@functools.partial(
    pl.kernel,
    out_type=jax.ShapeDtypeStruct((2 * NA, CO), jnp.float32),
    mesh=_mesh(),
    scratch_types=[
        pltpu.VMEM((RPC, 128), jnp.int32),
        pltpu.VMEM((CH, CO), jnp.float32),
        pltpu.VMEM_SHARED((NA, CO), jnp.float32),
    ],
    compiler_params=_sc_params,
)
def _scatter_k(msg, dst2, zeros, out, idx_v, msg_v, acc):
  _scatter_body(msg, dst2, zeros, out, idx_v, msg_v, acc)


@functools.partial(
    pl.kernel,
    out_type=jax.ShapeDtypeStruct((4 * NA, CO), jnp.float32),
    mesh=_mesh(),
    scratch_types=[
        pltpu.VMEM((RPC, 128), jnp.int32),
        pltpu.VMEM((CH, CO), jnp.float32),
        pltpu.VMEM((128, CO), jnp.float32),
        pltpu.VMEM_SHARED((NA, CO), jnp.float32),
        pltpu.VMEM_SHARED((NA, CO), jnp.float32),
    ],
    compiler_params=_sc_params,
)
def _scatter_deg_k(msg, dst2, zeros, ones, out, idx_v, msg_v, ones_v,
                   acc, acc_d):
  cid = lax.axis_index("c")
  sid = lax.axis_index("s")
  wid = sid * NC + cid
  ebase = wid * EPW
  rbase = wid * ROWS_PW

  pltpu.sync_copy(zeros.at[pl.ds(sid * NPS, NPS)],
                  acc.at[pl.ds(sid * NPS, NPS)])
  pltpu.sync_copy(zeros.at[pl.ds(sid * NPS, NPS)],
                  acc_d.at[pl.ds(sid * NPS, NPS)])
  pltpu.sync_copy(ones, ones_v)
  plsc.subcore_barrier()

  def step(i, carry):
    pltpu.sync_copy(dst2.at[pl.ds(rbase + i * RPC, RPC)], idx_v)
    pltpu.sync_copy(msg.at[pl.ds(ebase + i * CH, CH)], msg_v)
    for j in range(RPC):
      pltpu.sync_copy(msg_v.at[pl.ds(j * 128, 128)],
                      acc.at[idx_v.at[j]], add=True)
      pltpu.sync_copy(ones_v, acc_d.at[idx_v.at[j]], add=True)
    return carry

  lax.fori_loop(0, NCHUNK, step, 0)
  plsc.subcore_barrier()
  pltpu.sync_copy(acc.at[pl.ds(sid * NPS, NPS)],
                  out.at[pl.ds(cid * NA + sid * NPS, NPS)])
  pltpu.sync_copy(acc_d.at[pl.ds(sid * NPS, NPS)],
                  out.at[pl.ds((2 + cid) * NA + sid * NPS, NPS)])


def _msg_body(post_ref, xj_ref, w_ref, out_ref):
  t = post_ref[...]
  t = t - jnp.floor(t)
  t0 = t[0:1, :]
  t1 = t[1:2, :]

  def fs(tt):
    return (0.5 * tt * tt - tt + 0.5, -tt * tt + tt + 0.5, 0.5 * tt * tt)

  f0 = fs(t0)
  f1 = fs(t1)
  xjt = xj_ref[...].T
  rows = []
  for a in range(3):
    for b in range(3):
      rows.append((f1[a] * f0[b]) * xjt)
  amat = jnp.concatenate(rows, axis=0)
  msgt = jnp.dot(w_ref[...], amat,
                 preferred_element_type=jnp.float32)
  out_ref[...] = msgt.T


@functools.lru_cache(maxsize=None)
def _make_msg(ci):
  return pl.pallas_call(
      _msg_body,
      grid=(EP // BE,),
      in_specs=[
          pl.BlockSpec((2, BE), lambda i: (0, i)),
          pl.BlockSpec((BE, ci), lambda i: (i, 0)),
          pl.BlockSpec((CO, S * ci), lambda i: (0, 0)),
      ],
      out_specs=pl.BlockSpec((BE, CO), lambda i: (i, 0)),
      out_shape=jax.ShapeDtypeStruct((EP, CO), jnp.float32),
  )


def _dinv_body(d0_ref, d1_ref, out_ref):
  deg = d0_ref[...] + d1_ref[...]
  out_ref[...] = 1.0 / jnp.maximum(deg, 1.0)


_dinv_k = pl.pallas_call(
    _dinv_body,
    grid=(NA // BN,),
    in_specs=[
        pl.BlockSpec((BN, CO), lambda i: (i + 2 * (NA // BN), 0)),
        pl.BlockSpec((BN, CO), lambda i: (i + 3 * (NA // BN), 0)),
    ],
    out_specs=pl.BlockSpec((BN, CO), lambda i: (i, 0)),
    out_shape=jax.ShapeDtypeStruct((NA, CO), jnp.float32),
)


def _comb_body(p0_ref, p1_ref, dinv_ref, x_ref, root_ref, b_ref, out_ref):
  agg = (p0_ref[...] + p1_ref[...]) * dinv_ref[...]
  o = agg + jnp.dot(x_ref[...], root_ref[...],
                    preferred_element_type=jnp.float32) + b_ref[...]
  out_ref[...] = jnp.minimum(jnp.maximum(o, 0.0), 6.0)


@functools.lru_cache(maxsize=None)
def _make_combine(ci):
  return pl.pallas_call(
      _comb_body,
      grid=(NA // BN,),
      in_specs=[
          pl.BlockSpec((BN, CO), lambda i: (i, 0)),
          pl.BlockSpec((BN, CO), lambda i: (i + NA // BN, 0)),
          pl.BlockSpec((BN, CO), lambda i: (i, 0)),
          pl.BlockSpec((BN, ci), lambda i: (i, 0)),
          pl.BlockSpec((ci, CO), lambda i: (0, 0)),
          pl.BlockSpec((1, CO), lambda i: (0, 0)),
      ],
      out_specs=pl.BlockSpec((BN, CO), lambda i: (i, 0)),
      out_shape=jax.ShapeDtypeStruct((NA, CO), jnp.float32),
  )


def _comb_cat_body(p0_ref, p1_ref, dinv_ref, x_ref, root_ref, b_ref,
                   skip_ref, out_ref):
  agg = (p0_ref[...] + p1_ref[...]) * dinv_ref[...]
  o = agg + jnp.dot(x_ref[...], root_ref[...],
                    preferred_element_type=jnp.float32) + b_ref[...]
  o = jnp.minimum(jnp.maximum(o, 0.0), 6.0)
  out_ref[...] = jnp.concatenate([o, skip_ref[...]], axis=1)


@functools.lru_cache(maxsize=None)
def _make_combine_cat(ci):
  return pl.pallas_call(
      _comb_cat_body,
      grid=(NA // BN,),
      in_specs=[
          pl.BlockSpec((BN, CO), lambda i: (i, 0)),
          pl.BlockSpec((BN, CO), lambda i: (i + NA // BN, 0)),
          pl.BlockSpec((BN, CO), lambda i: (i, 0)),
          pl.BlockSpec((BN, ci), lambda i: (i, 0)),
          pl.BlockSpec((ci, CO), lambda i: (0, 0)),
          pl.BlockSpec((1, CO), lambda i: (0, 0)),
          pl.BlockSpec((BN, CO), lambda i: (i, 0)),
      ],
      out_specs=pl.BlockSpec((BN, 2 * CO), lambda i: (i, 0)),
      out_shape=jax.ShapeDtypeStruct((NA, 2 * CO), jnp.float32),
  )


def _final_body(flat_ref, w_ref, b_ref, batch_ref, out_ref):
  m = jnp.max(batch_ref[...])
  delta = (m + 1 - BATCH).astype(jnp.float32)
  out_ref[...] = jnp.dot(flat_ref[...], w_ref[...],
                         preferred_element_type=jnp.float32) \
      + b_ref[...] + delta


_final_k = pl.pallas_call(
    _final_body,
    out_shape=jax.ShapeDtypeStruct((BATCH, LIN_OUT), jnp.float32),
)


def kernel(x, edge_index, edge_attr, batch, pos, params):
  del edge_attr
  f32 = jnp.float32
  src = edge_index[0]
  dst = edge_index[1]
  pad_e = EP - E
  src2 = jnp.concatenate(
      [src, jnp.zeros((pad_e,), jnp.int32)]).reshape(EP // 128, 128)
  dst2 = jnp.concatenate(
      [dst, jnp.full((pad_e,), N, jnp.int32)]).reshape(EP // 128, 128)
  posp = jnp.concatenate([pos, jnp.zeros((pad_e, 2), f32)], axis=0)
  zeros_n = jnp.zeros((NA, CO), f32)
  ones_sc = jnp.ones((128, CO), f32)

  xpad = jnp.zeros((NA, CO), f32).at[:N].set(x)

  def wmat(l, ci):
    return params['conv%d_w' % l].transpose(2, 0, 1).reshape(CO, S * ci)

  post = posp.T

  def spmm(h, ci, l):
    xj = _make_gather(NA, ci)(h, src2)
    msg = _make_msg(ci)(post, xj, wmat(l, ci))
    return _scatter_k(msg, dst2, zeros_n)

  def cargs(l):
    return params['conv%d_root' % l], params['conv%d_b' % l].reshape(1, CO)

  xj = _make_gather(NA, CO)(xpad, src2)
  msg = _make_msg(CO)(post, xj, wmat(0, CO))
  p = _scatter_deg_k(msg, dst2, zeros_n, ones_sc)
  dinv = _dinv_k(p, p)
  o1 = _make_combine(CO)(p, p, dinv, xpad, *cargs(0))
  p = spmm(o1, CO, 1)
  o2 = _make_combine(CO)(p, p, dinv, o1, *cargs(1))
  p = spmm(o2, CO, 2)
  cat32 = _make_combine_cat(CO)(p, p, dinv, o2, *cargs(2), o2)
  p = spmm(cat32, 2 * CO, 3)
  cat32 = _make_combine_cat(2 * CO)(p, p, dinv, cat32, *cargs(3), o1)
  p = spmm(cat32, 2 * CO, 4)
  d = _make_combine(2 * CO)(p, p, dinv, cat32, *cargs(4))

  flat = d[:N].reshape(BATCH, LIN_IN)
  batch2 = batch.reshape(BATCH, N // BATCH)
  return _final_k(flat, params['lin_w'],
                  params['lin_b'].reshape(1, LIN_OUT), batch2)


# speedup vs baseline: 4.6247x; 2.1480x over previous
import functools

import jax
import jax.numpy as jnp
from jax import lax
from jax.experimental import pallas as pl
from jax.experimental.pallas import tpu as pltpu
from jax.experimental.pallas import tpu_sc as plsc

N = 50000
E = 800000
S = 9
CO = 16
BATCH = 100
LIN_IN = 500 * 16
LIN_OUT = 8

NC = 2
NS = 16
NW = NC * NS

NA = 51200
EP = 819200
EPW = EP // NW
CH = 1024
NCHUNK = EPW // CH
RPC = CH // 128
ROWS_PW = EPW // 128
NPS = NA // NS

BE = 2048
BN = 2048

_mesh = functools.partial(
    plsc.VectorSubcoreMesh, core_axis_name="c", subcore_axis_name="s")
_sc_params = pltpu.CompilerParams(use_tc_tiling_on_sc=False)


@functools.lru_cache(maxsize=None)
def _make_gather(nparts):
  @functools.partial(
      pl.kernel,
      out_type=[jax.ShapeDtypeStruct((EP, CO), jnp.float32)] * nparts,
      mesh=_mesh(),
      scratch_types=[
          pltpu.VMEM((RPC, 128), jnp.int32),
          *([pltpu.VMEM((CH, CO), jnp.float32)] * nparts),
          pltpu.SemaphoreType.DMA,
      ],
      compiler_params=_sc_params,
  )
  def gather_k(*refs):
    tables = refs[:nparts]
    src2 = refs[nparts]
    outs = refs[nparts + 1:2 * nparts + 1]
    idx_v = refs[2 * nparts + 1]
    rows = refs[2 * nparts + 2:3 * nparts + 2]
    sem = refs[3 * nparts + 2]
    cid = lax.axis_index("c")
    sid = lax.axis_index("s")
    wid = sid * NC + cid
    ebase = wid * EPW
    rbase = wid * ROWS_PW

    def step(i, carry):
      pltpu.sync_copy(src2.at[pl.ds(rbase + i * RPC, RPC)], idx_v)
      cps = [
          pltpu.async_copy(t.at[idx_v.at[j]],
                           rv.at[pl.ds(j * 128, 128)], sem)
          for t, rv in zip(tables, rows)
          for j in range(RPC)
      ]
      for cp in cps:
        cp.wait()
      for rv, out in zip(rows, outs):
        pltpu.sync_copy(rv, out.at[pl.ds(ebase + i * CH, CH)])
      return carry

    lax.fori_loop(0, NCHUNK, step, 0)

  return gather_k


def _scatter_body(msg, dst2, zeros, out, idx_v, msg_v, acc):
  cid = lax.axis_index("c")
  sid = lax.axis_index("s")
  wid = sid * NC + cid
  ebase = wid * EPW
  rbase = wid * ROWS_PW

  pltpu.sync_copy(zeros.at[pl.ds(sid * NPS, NPS)],
                  acc.at[pl.ds(sid * NPS, NPS)])
  plsc.subcore_barrier()

  def step(i, carry):
    pltpu.sync_copy(dst2.at[pl.ds(rbase + i * RPC, RPC)], idx_v)
    pltpu.sync_copy(msg.at[pl.ds(ebase + i * CH, CH)], msg_v)
    for j in range(RPC):
      pltpu.sync_copy(msg_v.at[pl.ds(j * 128, 128)],
                      acc.at[idx_v.at[j]], add=True)
    return carry

  lax.fori_loop(0, NCHUNK, step, 0)
  plsc.subcore_barrier()
  pltpu.sync_copy(acc.at[pl.ds(sid * NPS, NPS)],
                  out.at[pl.ds(cid * NA + sid * NPS, NPS)])


@functools.partial(
    pl.kernel,
    out_type=jax.ShapeDtypeStruct((2 * NA, CO), jnp.float32),
    mesh=_mesh(),
    scratch_types=[
        pltpu.VMEM((RPC, 128), jnp.int32),
        pltpu.VMEM((CH, CO), jnp.float32),
        pltpu.VMEM_SHARED((NA, CO), jnp.float32),
    ],
    compiler_params=_sc_params,
)
def _scatter_k(msg, dst2, zeros, out, idx_v, msg_v, acc):
  _scatter_body(msg, dst2, zeros, out, idx_v, msg_v, acc)


@functools.partial(
    pl.kernel,
    out_type=jax.ShapeDtypeStruct((4 * NA, CO), jnp.float32),
    mesh=_mesh(),
    scratch_types=[
        pltpu.VMEM((RPC, 128), jnp.int32),
        pltpu.VMEM((CH, CO), jnp.float32),
        pltpu.VMEM((128, CO), jnp.float32),
        pltpu.VMEM_SHARED((NA, CO), jnp.float32),
        pltpu.VMEM_SHARED((NA, CO), jnp.float32),
    ],
    compiler_params=_sc_params,
)
def _scatter_deg_k(msg, dst2, zeros, ones, out, idx_v, msg_v, ones_v,
                   acc, acc_d):
  cid = lax.axis_index("c")
  sid = lax.axis_index("s")
  wid = sid * NC + cid
  ebase = wid * EPW
  rbase = wid * ROWS_PW

  pltpu.sync_copy(zeros.at[pl.ds(sid * NPS, NPS)],
                  acc.at[pl.ds(sid * NPS, NPS)])
  pltpu.sync_copy(zeros.at[pl.ds(sid * NPS, NPS)],
                  acc_d.at[pl.ds(sid * NPS, NPS)])
  pltpu.sync_copy(ones, ones_v)
  plsc.subcore_barrier()

  def step(i, carry):
    pltpu.sync_copy(dst2.at[pl.ds(rbase + i * RPC, RPC)], idx_v)
    pltpu.sync_copy(msg.at[pl.ds(ebase + i * CH, CH)], msg_v)
    for j in range(RPC):
      pltpu.sync_copy(msg_v.at[pl.ds(j * 128, 128)],
                      acc.at[idx_v.at[j]], add=True)
      pltpu.sync_copy(ones_v, acc_d.at[idx_v.at[j]], add=True)
    return carry

  lax.fori_loop(0, NCHUNK, step, 0)
  plsc.subcore_barrier()
  pltpu.sync_copy(acc.at[pl.ds(sid * NPS, NPS)],
                  out.at[pl.ds(cid * NA + sid * NPS, NPS)])
  pltpu.sync_copy(acc_d.at[pl.ds(sid * NPS, NPS)],
                  out.at[pl.ds((2 + cid) * NA + sid * NPS, NPS)])


PPE = BE // 8


def _msg_body_n(nparts, refs):
  post_ref = refs[0]
  xj_refs = refs[1:1 + nparts]
  w_ref = refs[1 + nparts]
  out_ref = refs[2 + nparts]
  t = post_ref[...]
  t = t - jnp.floor(t)
  t0 = t[0:1, :]
  t1 = t[1:2, :]

  def fs(tt):
    return (0.5 * tt * tt - tt + 0.5, -tt * tt + tt + 0.5, 0.5 * tt * tt)

  f0 = fs(t0)
  f1 = fs(t1)
  bb = [f1[a] * f0[b] for a in range(3) for b in range(3)]
  pts = [r[...].T for r in xj_refs]
  w = w_ref[...]
  mrows = []
  for j in range(8):
    xjt = jnp.concatenate(
        [pt[CO * j:CO * (j + 1), :] for pt in pts], axis=0)
    amat = jnp.concatenate(
        [bbs[:, PPE * j:PPE * (j + 1)] * xjt for bbs in bb], axis=0)
    mrows.append(jnp.dot(w, amat, preferred_element_type=jnp.float32))
  out_ref[...] = jnp.concatenate(mrows, axis=0).T


@functools.lru_cache(maxsize=None)
def _make_msg(ci):
  nparts = ci // CO
  xj_specs = [pl.BlockSpec((BE * CO // 128, 128), lambda i: (i, 0))
              for _ in range(nparts)]
  return pl.pallas_call(
      lambda *refs: _msg_body_n(nparts, refs),
      grid=(EP // BE,),
      in_specs=[
          pl.BlockSpec((2, BE), lambda i: (0, i)),
          *xj_specs,
          pl.BlockSpec((CO, S * ci), lambda i: (0, 0)),
      ],
      out_specs=pl.BlockSpec((BE * CO // 128, 128), lambda i: (i, 0)),
      out_shape=jax.ShapeDtypeStruct((EP * CO // 128, 128), jnp.float32),
  )


def _dinv_body(d0_ref, d1_ref, out_ref):
  deg = d0_ref[...] + d1_ref[...]
  out_ref[...] = 1.0 / jnp.maximum(deg, 1.0)


PBN = BN * CO // 128
PNA = NA * CO // 128

_dinv_k = pl.pallas_call(
    _dinv_body,
    grid=(NA // BN,),
    in_specs=[
        pl.BlockSpec((PBN, 128), lambda i: (i + 2 * (NA // BN), 0)),
        pl.BlockSpec((PBN, 128), lambda i: (i + 3 * (NA // BN), 0)),
    ],
    out_specs=pl.BlockSpec((PBN, 128), lambda i: (i, 0)),
    out_shape=jax.ShapeDtypeStruct((PNA, 128), jnp.float32),
)


def _comb_body_n(nparts, refs):
  p0_ref, p1_ref, dinv_ref = refs[0], refs[1], refs[2]
  x_refs = refs[3:3 + nparts]
  rk_refs = refs[3 + nparts:3 + 2 * nparts]
  b_ref = refs[3 + 2 * nparts]
  out_ref = refs[4 + 2 * nparts]
  agg = (p0_ref[...] + p1_ref[...]) * dinv_ref[...]
  xr = b_ref[...]
  for xref, rkref in zip(x_refs, rk_refs):
    xr = xr + jnp.dot(xref[...], rkref[...],
                      preferred_element_type=jnp.float32)
  o = agg + xr
  out_ref[...] = jnp.minimum(jnp.maximum(o, 0.0), 6.0)


@functools.lru_cache(maxsize=None)
def _make_combine(nparts):
  x_specs = [pl.BlockSpec((PBN, 128), lambda i: (i, 0))
             for _ in range(nparts)]
  rk_specs = [pl.BlockSpec((128, 128), lambda i: (0, 0))
              for _ in range(nparts)]
  return pl.pallas_call(
      lambda *refs: _comb_body_n(nparts, refs),
      grid=(NA // BN,),
      in_specs=[
          pl.BlockSpec((PBN, 128), lambda i: (i, 0)),
          pl.BlockSpec((PBN, 128), lambda i: (i + NA // BN, 0)),
          pl.BlockSpec((PBN, 128), lambda i: (i, 0)),
          *x_specs,
          *rk_specs,
          pl.BlockSpec((1, 128), lambda i: (0, 0)),
      ],
      out_specs=pl.BlockSpec((PBN, 128), lambda i: (i, 0)),
      out_shape=jax.ShapeDtypeStruct((PNA, 128), jnp.float32),
  )


def _final_body(flat_ref, w_ref, b_ref, batch_ref, out_ref):
  m = jnp.max(batch_ref[...])
  delta = (m + 1 - BATCH).astype(jnp.float32)
  out_ref[...] = jnp.dot(flat_ref[...], w_ref[...],
                         preferred_element_type=jnp.float32) \
      + b_ref[...] + delta


_final_k = pl.pallas_call(
    _final_body,
    out_shape=jax.ShapeDtypeStruct((BATCH, LIN_OUT), jnp.float32),
)


def kernel(x, edge_index, edge_attr, batch, pos, params):
  del edge_attr
  f32 = jnp.float32
  src = edge_index[0]
  dst = edge_index[1]
  pad_e = EP - E
  src2 = jnp.concatenate(
      [src, jnp.zeros((pad_e,), jnp.int32)]).reshape(EP // 128, 128)
  dst2 = jnp.concatenate(
      [dst, jnp.full((pad_e,), N, jnp.int32)]).reshape(EP // 128, 128)
  posp = jnp.concatenate([pos, jnp.zeros((pad_e, 2), f32)], axis=0)
  zeros_n = jnp.zeros((NA, CO), f32)
  ones_sc = jnp.ones((128, CO), f32)

  xpad = jnp.zeros((NA, CO), f32).at[:N].set(x)
  xpad_p = xpad.reshape(PNA, 128)

  def wmat(l, ci):
    return params['conv%d_w' % l].transpose(2, 0, 1).reshape(CO, S * ci)

  post = posp.reshape(EP // BE, PPE, 8, 2).transpose(0, 2, 1, 3) \
      .reshape(EP, 2).T

  eye8 = jnp.eye(8, dtype=f32)

  def kron8(m):
    return jnp.kron(eye8, m)

  def rk(l, nparts):
    root = params['conv%d_root' % l]
    return [kron8(root[k * CO:(k + 1) * CO]) for k in range(nparts)]

  def bt(l):
    return jnp.tile(params['conv%d_b' % l].reshape(1, CO), (1, 8))

  def gather_p(hs):
    lins = [h.reshape(NA, CO) for h in hs]
    outs = _make_gather(len(hs))(*lins, src2)
    if not isinstance(outs, (list, tuple)):
      outs = [outs]
    return [o.reshape(EP * CO // 128, 128) for o in outs]

  def spmm(hs, l):
    xjs = gather_p(hs)
    msg = _make_msg(CO * len(hs))(post, *xjs, wmat(l, CO * len(hs)))
    p = _scatter_k(msg.reshape(EP, CO), dst2, zeros_n)
    return p.reshape(2 * PNA, 128)

  xjs = gather_p([xpad_p])
  msg = _make_msg(CO)(post, xjs[0], wmat(0, CO))
  p = _scatter_deg_k(msg.reshape(EP, CO), dst2, zeros_n, ones_sc)
  p = p.reshape(4 * PNA, 128)
  dinv = _dinv_k(p, p)
  o1 = _make_combine(1)(p, p, dinv, xpad_p, *rk(0, 1), bt(0))
  p = spmm([o1], 1)
  o2 = _make_combine(1)(p, p, dinv, o1, *rk(1, 1), bt(1))
  p = spmm([o2], 2)
  o3 = _make_combine(1)(p, p, dinv, o2, *rk(2, 1), bt(2))
  p = spmm([o3, o2], 3)
  d3 = _make_combine(2)(p, p, dinv, o3, o2, *rk(3, 2), bt(3))
  p = spmm([d3, o1], 4)
  d = _make_combine(2)(p, p, dinv, d3, o1, *rk(4, 2), bt(4))

  flat = jax.lax.slice(d, (0, 0), (N * CO // 128, 128)).reshape(BATCH, LIN_IN)
  batch2 = batch.reshape(BATCH, N // BATCH)
  return _final_k(flat, params['lin_w'],
                  params['lin_b'].reshape(1, LIN_OUT), batch2)


# speedup vs baseline: 5.8984x; 1.2754x over previous
import functools

import jax
import jax.numpy as jnp
from jax import lax
from jax.experimental import pallas as pl
from jax.experimental.pallas import tpu as pltpu
from jax.experimental.pallas import tpu_sc as plsc

N = 50000
E = 800000
S = 9
CO = 16
BATCH = 100
LIN_IN = 500 * 16
LIN_OUT = 8

NC = 2
NS = 16
NW = NC * NS

NA = 51200
EP = 819200
EPW = EP // NW
CH = 1024
NCHUNK = EPW // CH
RPC = CH // 128
ROWS_PW = EPW // 128
NPS = NA // NS

GCH = 1280
GNCHUNK = EPW // GCH
GRPC = GCH // 128

BE = 4096
BN = 2048

_mesh = functools.partial(
    plsc.VectorSubcoreMesh, core_axis_name="c", subcore_axis_name="s")
_sc_params = pltpu.CompilerParams(use_tc_tiling_on_sc=False)


@functools.lru_cache(maxsize=None)
def _make_gather(nparts):
  @functools.partial(
      pl.kernel,
      out_type=[jax.ShapeDtypeStruct((EP, CO), jnp.float32)] * nparts,
      mesh=_mesh(),
      scratch_types=[
          *([pltpu.VMEM((GRPC, 128), jnp.int32)] * 2),
          *([pltpu.VMEM((GCH, CO), jnp.float32)] * (2 * nparts)),
          pltpu.SemaphoreType.DMA,
          pltpu.SemaphoreType.DMA,
      ],
      compiler_params=_sc_params,
  )
  def gather_k(*refs):
    tables = refs[:nparts]
    src2 = refs[nparts]
    outs = refs[nparts + 1:2 * nparts + 1]
    idx_v = refs[2 * nparts + 1:2 * nparts + 3]
    rows = refs[2 * nparts + 3:4 * nparts + 3]
    sems = refs[4 * nparts + 3:4 * nparts + 5]
    cid = lax.axis_index("c")
    sid = lax.axis_index("s")
    wid = sid * NC + cid
    ebase = wid * EPW
    rbase = wid * ROWS_PW

    def fire(k, b):
      pltpu.sync_copy(src2.at[pl.ds(rbase + k * GRPC, GRPC)], idx_v[b])
      return [
          pltpu.async_copy(t.at[idx_v[b].at[j]],
                           rows[b * nparts + pi].at[pl.ds(j * 128, 128)],
                           sems[b])
          for pi, t in enumerate(tables)
          for j in range(GRPC)
      ]

    cps = {0: fire(0, 0)}
    for k in range(GNCHUNK):
      b = k % 2
      if k + 1 < GNCHUNK:
        cps[k + 1] = fire(k + 1, 1 - b)
      for cp in cps.pop(k):
        cp.wait()
      for pi, out in enumerate(outs):
        pltpu.sync_copy(rows[b * nparts + pi],
                        out.at[pl.ds(ebase + k * GCH, GCH)])

  return gather_k


def _scatter_body(msg, dst2, zeros, out, idx_v, msg_v, acc):
  cid = lax.axis_index("c")
  sid = lax.axis_index("s")
  wid = sid * NC + cid
  ebase = wid * EPW
  rbase = wid * ROWS_PW

  pltpu.sync_copy(zeros.at[pl.ds(sid * NPS, NPS)],
                  acc.at[pl.ds(sid * NPS, NPS)])
  plsc.subcore_barrier()

  def step(i, carry):
    pltpu.sync_copy(dst2.at[pl.ds(rbase + i * RPC, RPC)], idx_v)
    pltpu.sync_copy(msg.at[pl.ds(ebase + i * CH, CH)], msg_v)
    for j in range(RPC):
      pltpu.sync_copy(msg_v.at[pl.ds(j * 128, 128)],
                      acc.at[idx_v.at[j]], add=True)
    return carry

  lax.fori_loop(0, NCHUNK, step, 0)
  plsc.subcore_barrier()
  pltpu.sync_copy(acc.at[pl.ds(sid * NPS, NPS)],
                  out.at[pl.ds(cid * NA + sid * NPS, NPS)])


@functools.partial(
    pl.kernel,
    out_type=jax.ShapeDtypeStruct((2 * NA, CO), jnp.float32),
    mesh=_mesh(),
    scratch_types=[
        pltpu.VMEM((RPC, 128), jnp.int32),
        pltpu.VMEM((CH, CO), jnp.float32),
        pltpu.VMEM_SHARED((NA, CO), jnp.float32),
    ],
    compiler_params=_sc_params,
)
def _scatter_k(msg, dst2, zeros, out, idx_v, msg_v, acc):
  _scatter_body(msg, dst2, zeros, out, idx_v, msg_v, acc)


@functools.partial(
    pl.kernel,
    out_type=jax.ShapeDtypeStruct((4 * NA, CO), jnp.float32),
    mesh=_mesh(),
    scratch_types=[
        pltpu.VMEM((RPC, 128), jnp.int32),
        pltpu.VMEM((CH, CO), jnp.float32),
        pltpu.VMEM((128, CO), jnp.float32),
        pltpu.VMEM_SHARED((NA, CO), jnp.float32),
        pltpu.VMEM_SHARED((NA, CO), jnp.float32),
    ],
    compiler_params=_sc_params,
)
def _scatter_deg_k(msg, dst2, zeros, ones, out, idx_v, msg_v, ones_v,
                   acc, acc_d):
  cid = lax.axis_index("c")
  sid = lax.axis_index("s")
  wid = sid * NC + cid
  ebase = wid * EPW
  rbase = wid * ROWS_PW

  pltpu.sync_copy(zeros.at[pl.ds(sid * NPS, NPS)],
                  acc.at[pl.ds(sid * NPS, NPS)])
  pltpu.sync_copy(zeros.at[pl.ds(sid * NPS, NPS)],
                  acc_d.at[pl.ds(sid * NPS, NPS)])
  pltpu.sync_copy(ones, ones_v)
  plsc.subcore_barrier()

  def step(i, carry):
    pltpu.sync_copy(dst2.at[pl.ds(rbase + i * RPC, RPC)], idx_v)
    pltpu.sync_copy(msg.at[pl.ds(ebase + i * CH, CH)], msg_v)
    for j in range(RPC):
      pltpu.sync_copy(msg_v.at[pl.ds(j * 128, 128)],
                      acc.at[idx_v.at[j]], add=True)
      pltpu.sync_copy(ones_v, acc_d.at[idx_v.at[j]], add=True)
    return carry

  lax.fori_loop(0, NCHUNK, step, 0)
  plsc.subcore_barrier()
  pltpu.sync_copy(acc.at[pl.ds(sid * NPS, NPS)],
                  out.at[pl.ds(cid * NA + sid * NPS, NPS)])
  pltpu.sync_copy(acc_d.at[pl.ds(sid * NPS, NPS)],
                  out.at[pl.ds((2 + cid) * NA + sid * NPS, NPS)])


PPE = BE // 8


def _msg_body_n(nparts, refs):
  post_ref = refs[0]
  xj_refs = refs[1:1 + nparts]
  w_ref = refs[1 + nparts]
  out_ref = refs[2 + nparts]
  t = post_ref[...]
  t = t - jnp.floor(t)
  t0 = t[0:1, :]
  t1 = t[1:2, :]

  def fs(tt):
    return (0.5 * tt * tt - tt + 0.5, -tt * tt + tt + 0.5, 0.5 * tt * tt)

  f0 = fs(t0)
  f1 = fs(t1)
  bb = [f1[a] * f0[b] for a in range(3) for b in range(3)]
  pts = [r[...].T for r in xj_refs]
  w = w_ref[...]
  mrows = []
  for j in range(8):
    xjt = jnp.concatenate(
        [pt[CO * j:CO * (j + 1), :] for pt in pts], axis=0)
    amat = jnp.concatenate(
        [bbs[:, PPE * j:PPE * (j + 1)] * xjt for bbs in bb], axis=0)
    mrows.append(jnp.dot(w, amat, preferred_element_type=jnp.float32))
  out_ref[...] = jnp.concatenate(mrows, axis=0).T


@functools.lru_cache(maxsize=None)
def _make_msg(ci):
  nparts = ci // CO
  xj_specs = [pl.BlockSpec((BE * CO // 128, 128), lambda i: (i, 0))
              for _ in range(nparts)]
  return pl.pallas_call(
      lambda *refs: _msg_body_n(nparts, refs),
      grid=(EP // BE,),
      in_specs=[
          pl.BlockSpec((2, BE), lambda i: (0, i)),
          *xj_specs,
          pl.BlockSpec((CO, S * ci), lambda i: (0, 0)),
      ],
      out_specs=pl.BlockSpec((BE * CO // 128, 128), lambda i: (i, 0)),
      out_shape=jax.ShapeDtypeStruct((EP * CO // 128, 128), jnp.float32),
  )


def _dinv_body(d0_ref, d1_ref, out_ref):
  deg = d0_ref[...] + d1_ref[...]
  out_ref[...] = 1.0 / jnp.maximum(deg, 1.0)


PBN = BN * CO // 128
PNA = NA * CO // 128

_dinv_k = pl.pallas_call(
    _dinv_body,
    grid=(NA // BN,),
    in_specs=[
        pl.BlockSpec((PBN, 128), lambda i: (i + 2 * (NA // BN), 0)),
        pl.BlockSpec((PBN, 128), lambda i: (i + 3 * (NA // BN), 0)),
    ],
    out_specs=pl.BlockSpec((PBN, 128), lambda i: (i, 0)),
    out_shape=jax.ShapeDtypeStruct((PNA, 128), jnp.float32),
)


def _comb_body_n(nparts, refs):
  p0_ref, p1_ref, dinv_ref = refs[0], refs[1], refs[2]
  x_refs = refs[3:3 + nparts]
  rk_refs = refs[3 + nparts:3 + 2 * nparts]
  b_ref = refs[3 + 2 * nparts]
  out_ref = refs[4 + 2 * nparts]
  agg = (p0_ref[...] + p1_ref[...]) * dinv_ref[...]
  xr = b_ref[...]
  for xref, rkref in zip(x_refs, rk_refs):
    xr = xr + jnp.dot(xref[...], rkref[...],
                      preferred_element_type=jnp.float32)
  o = agg + xr
  out_ref[...] = jnp.minimum(jnp.maximum(o, 0.0), 6.0)


@functools.lru_cache(maxsize=None)
def _make_combine(nparts):
  x_specs = [pl.BlockSpec((PBN, 128), lambda i: (i, 0))
             for _ in range(nparts)]
  rk_specs = [pl.BlockSpec((128, 128), lambda i: (0, 0))
              for _ in range(nparts)]
  return pl.pallas_call(
      lambda *refs: _comb_body_n(nparts, refs),
      grid=(NA // BN,),
      in_specs=[
          pl.BlockSpec((PBN, 128), lambda i: (i, 0)),
          pl.BlockSpec((PBN, 128), lambda i: (i + NA // BN, 0)),
          pl.BlockSpec((PBN, 128), lambda i: (i, 0)),
          *x_specs,
          *rk_specs,
          pl.BlockSpec((1, 128), lambda i: (0, 0)),
      ],
      out_specs=pl.BlockSpec((PBN, 128), lambda i: (i, 0)),
      out_shape=jax.ShapeDtypeStruct((PNA, 128), jnp.float32),
  )


def _final_body(flat_ref, w_ref, b_ref, batch_ref, out_ref):
  m = jnp.max(batch_ref[...])
  delta = (m + 1 - BATCH).astype(jnp.float32)
  out_ref[...] = jnp.dot(flat_ref[...], w_ref[...],
                         preferred_element_type=jnp.float32) \
      + b_ref[...] + delta


_final_k = pl.pallas_call(
    _final_body,
    out_shape=jax.ShapeDtypeStruct((BATCH, LIN_OUT), jnp.float32),
)


def kernel(x, edge_index, edge_attr, batch, pos, params):
  del edge_attr
  f32 = jnp.float32
  src = edge_index[0]
  dst = edge_index[1]
  pad_e = EP - E
  src2 = jnp.concatenate(
      [src, jnp.zeros((pad_e,), jnp.int32)]).reshape(EP // 128, 128)
  dst2 = jnp.concatenate(
      [dst, jnp.full((pad_e,), N, jnp.int32)]).reshape(EP // 128, 128)
  posp = jnp.concatenate([pos, jnp.zeros((pad_e, 2), f32)], axis=0)
  zeros_n = jnp.zeros((NA, CO), f32)
  ones_sc = jnp.ones((128, CO), f32)

  xpad = jnp.zeros((NA, CO), f32).at[:N].set(x)
  xpad_p = xpad.reshape(PNA, 128)

  def wmat(l, ci):
    return params['conv%d_w' % l].transpose(2, 0, 1).reshape(CO, S * ci)

  post = posp.reshape(EP // BE, PPE, 8, 2).transpose(0, 2, 1, 3) \
      .reshape(EP, 2).T

  eye8 = jnp.eye(8, dtype=f32)

  def kron8(m):
    return jnp.kron(eye8, m)

  def rk(l, nparts):
    root = params['conv%d_root' % l]
    return [kron8(root[k * CO:(k + 1) * CO]) for k in range(nparts)]

  def bt(l):
    return jnp.tile(params['conv%d_b' % l].reshape(1, CO), (1, 8))

  def gather_p(hs):
    lins = [h.reshape(NA, CO) for h in hs]
    outs = _make_gather(len(hs))(*lins, src2)
    if not isinstance(outs, (list, tuple)):
      outs = [outs]
    return [o.reshape(EP * CO // 128, 128) for o in outs]

  def spmm(hs, l):
    xjs = gather_p(hs)
    msg = _make_msg(CO * len(hs))(post, *xjs, wmat(l, CO * len(hs)))
    p = _scatter_k(msg.reshape(EP, CO), dst2, zeros_n)
    return p.reshape(2 * PNA, 128)

  xjs = gather_p([xpad_p])
  msg = _make_msg(CO)(post, xjs[0], wmat(0, CO))
  p = _scatter_deg_k(msg.reshape(EP, CO), dst2, zeros_n, ones_sc)
  p = p.reshape(4 * PNA, 128)
  dinv = _dinv_k(p, p)
  o1 = _make_combine(1)(p, p, dinv, xpad_p, *rk(0, 1), bt(0))
  p = spmm([o1], 1)
  o2 = _make_combine(1)(p, p, dinv, o1, *rk(1, 1), bt(1))
  p = spmm([o2], 2)
  o3 = _make_combine(1)(p, p, dinv, o2, *rk(2, 1), bt(2))
  p = spmm([o3, o2], 3)
  d3 = _make_combine(2)(p, p, dinv, o3, o2, *rk(3, 2), bt(3))
  p = spmm([d3, o1], 4)
  d = _make_combine(2)(p, p, dinv, d3, o1, *rk(4, 2), bt(4))

  flat = jax.lax.slice(d, (0, 0), (N * CO // 128, 128)).reshape(BATCH, LIN_IN)
  batch2 = batch.reshape(BATCH, N // BATCH)
  return _final_k(flat, params['lin_w'],
                  params['lin_b'].reshape(1, LIN_OUT), batch2)


# speedup vs baseline: 6.5751x; 1.1147x over previous
import functools

import jax
import jax.numpy as jnp
from jax import lax
from jax.experimental import pallas as pl
from jax.experimental.pallas import tpu as pltpu
from jax.experimental.pallas import tpu_sc as plsc

N = 50000
E = 800000
S = 9
CO = 16
BATCH = 100
LIN_IN = 500 * 16
LIN_OUT = 8

NC = 2
NS = 16
NW = NC * NS

NA = 51200
EP = 819200
EPW = EP // NW
CH = 1024
NCHUNK = EPW // CH
RPC = CH // 128
ROWS_PW = EPW // 128
NPS = NA // NS

GCH = 1280
GNCHUNK = EPW // GCH
GRPC = GCH // 128

BE = 8192
BN = 2048

_mesh = functools.partial(
    plsc.VectorSubcoreMesh, core_axis_name="c", subcore_axis_name="s")
_sc_params = pltpu.CompilerParams(use_tc_tiling_on_sc=False)


@functools.lru_cache(maxsize=None)
def _make_gather(nparts):
  @functools.partial(
      pl.kernel,
      out_type=[jax.ShapeDtypeStruct((EP, CO), jnp.float32)] * nparts,
      mesh=_mesh(),
      scratch_types=[
          *([pltpu.VMEM((GRPC, 128), jnp.int32)] * 2),
          *([pltpu.VMEM((GCH, CO), jnp.float32)] * (2 * nparts)),
          pltpu.SemaphoreType.DMA,
          pltpu.SemaphoreType.DMA,
      ],
      compiler_params=_sc_params,
  )
  def gather_k(*refs):
    tables = refs[:nparts]
    src2 = refs[nparts]
    outs = refs[nparts + 1:2 * nparts + 1]
    idx_v = refs[2 * nparts + 1:2 * nparts + 3]
    rows = refs[2 * nparts + 3:4 * nparts + 3]
    sems = refs[4 * nparts + 3:4 * nparts + 5]
    cid = lax.axis_index("c")
    sid = lax.axis_index("s")
    wid = sid * NC + cid
    ebase = wid * EPW
    rbase = wid * ROWS_PW

    def fire(k, b):
      pltpu.sync_copy(src2.at[pl.ds(rbase + k * GRPC, GRPC)], idx_v[b])
      return [
          pltpu.async_copy(t.at[idx_v[b].at[j]],
                           rows[b * nparts + pi].at[pl.ds(j * 128, 128)],
                           sems[b])
          for pi, t in enumerate(tables)
          for j in range(GRPC)
      ]

    cps = {0: fire(0, 0)}
    for k in range(GNCHUNK):
      b = k % 2
      if k + 1 < GNCHUNK:
        cps[k + 1] = fire(k + 1, 1 - b)
      for cp in cps.pop(k):
        cp.wait()
      for pi, out in enumerate(outs):
        pltpu.sync_copy(rows[b * nparts + pi],
                        out.at[pl.ds(ebase + k * GCH, GCH)])

  return gather_k


def _scatter_body(msg, dst2, zeros, out, idx_v, msg_v, acc):
  cid = lax.axis_index("c")
  sid = lax.axis_index("s")
  wid = sid * NC + cid
  ebase = wid * EPW
  rbase = wid * ROWS_PW

  pltpu.sync_copy(zeros.at[pl.ds(sid * NPS, NPS)],
                  acc.at[pl.ds(sid * NPS, NPS)])
  plsc.subcore_barrier()

  def step(i, carry):
    pltpu.sync_copy(dst2.at[pl.ds(rbase + i * RPC, RPC)], idx_v)
    pltpu.sync_copy(msg.at[pl.ds(ebase + i * CH, CH)], msg_v)
    for j in range(RPC):
      pltpu.sync_copy(msg_v.at[pl.ds(j * 128, 128)],
                      acc.at[idx_v.at[j]], add=True)
    return carry

  lax.fori_loop(0, NCHUNK, step, 0)
  plsc.subcore_barrier()
  pltpu.sync_copy(acc.at[pl.ds(sid * NPS, NPS)],
                  out.at[pl.ds(cid * NA + sid * NPS, NPS)])


@functools.partial(
    pl.kernel,
    out_type=jax.ShapeDtypeStruct((2 * NA, CO), jnp.float32),
    mesh=_mesh(),
    scratch_types=[
        pltpu.VMEM((RPC, 128), jnp.int32),
        pltpu.VMEM((CH, CO), jnp.float32),
        pltpu.VMEM_SHARED((NA, CO), jnp.float32),
    ],
    compiler_params=_sc_params,
)
def _scatter_k(msg, dst2, zeros, out, idx_v, msg_v, acc):
  _scatter_body(msg, dst2, zeros, out, idx_v, msg_v, acc)


@functools.partial(
    pl.kernel,
    out_type=jax.ShapeDtypeStruct((4 * NA, CO), jnp.float32),
    mesh=_mesh(),
    scratch_types=[
        pltpu.VMEM((RPC, 128), jnp.int32),
        pltpu.VMEM((CH, CO), jnp.float32),
        pltpu.VMEM((128, CO), jnp.float32),
        pltpu.VMEM_SHARED((NA, CO), jnp.float32),
        pltpu.VMEM_SHARED((NA, CO), jnp.float32),
    ],
    compiler_params=_sc_params,
)
def _scatter_deg_k(msg, dst2, zeros, ones, out, idx_v, msg_v, ones_v,
                   acc, acc_d):
  cid = lax.axis_index("c")
  sid = lax.axis_index("s")
  wid = sid * NC + cid
  ebase = wid * EPW
  rbase = wid * ROWS_PW

  pltpu.sync_copy(zeros.at[pl.ds(sid * NPS, NPS)],
                  acc.at[pl.ds(sid * NPS, NPS)])
  pltpu.sync_copy(zeros.at[pl.ds(sid * NPS, NPS)],
                  acc_d.at[pl.ds(sid * NPS, NPS)])
  pltpu.sync_copy(ones, ones_v)
  plsc.subcore_barrier()

  def step(i, carry):
    pltpu.sync_copy(dst2.at[pl.ds(rbase + i * RPC, RPC)], idx_v)
    pltpu.sync_copy(msg.at[pl.ds(ebase + i * CH, CH)], msg_v)
    for j in range(RPC):
      pltpu.sync_copy(msg_v.at[pl.ds(j * 128, 128)],
                      acc.at[idx_v.at[j]], add=True)
      pltpu.sync_copy(ones_v, acc_d.at[idx_v.at[j]], add=True)
    return carry

  lax.fori_loop(0, NCHUNK, step, 0)
  plsc.subcore_barrier()
  pltpu.sync_copy(acc.at[pl.ds(sid * NPS, NPS)],
                  out.at[pl.ds(cid * NA + sid * NPS, NPS)])
  pltpu.sync_copy(acc_d.at[pl.ds(sid * NPS, NPS)],
                  out.at[pl.ds((2 + cid) * NA + sid * NPS, NPS)])


PPE = BE // 8


def _msg_body_n(nparts, refs):
  post_ref = refs[0]
  xj_refs = refs[1:1 + nparts]
  w_ref = refs[1 + nparts]
  out_ref = refs[2 + nparts]
  t = post_ref[...]
  t = t - jnp.floor(t)
  t0 = t[0:1, :]
  t1 = t[1:2, :]

  def fs(tt):
    return (0.5 * tt * tt - tt + 0.5, -tt * tt + tt + 0.5, 0.5 * tt * tt)

  f0 = fs(t0)
  f1 = fs(t1)
  bb = [f1[a] * f0[b] for a in range(3) for b in range(3)]
  pts = [r[...].T for r in xj_refs]
  w = w_ref[...]
  mrows = []
  for j in range(8):
    xjt = jnp.concatenate(
        [pt[CO * j:CO * (j + 1), :] for pt in pts], axis=0)
    amat = jnp.concatenate(
        [bbs[:, PPE * j:PPE * (j + 1)] * xjt for bbs in bb], axis=0)
    mrows.append(jnp.dot(w, amat, preferred_element_type=jnp.float32))
  out_ref[...] = jnp.concatenate(mrows, axis=0).T


@functools.lru_cache(maxsize=None)
def _make_msg(ci):
  nparts = ci // CO
  xj_specs = [pl.BlockSpec((BE * CO // 128, 128), lambda i: (i, 0))
              for _ in range(nparts)]
  return pl.pallas_call(
      lambda *refs: _msg_body_n(nparts, refs),
      grid=(EP // BE,),
      in_specs=[
          pl.BlockSpec((2, BE), lambda i: (0, i)),
          *xj_specs,
          pl.BlockSpec((CO, S * ci), lambda i: (0, 0)),
      ],
      out_specs=pl.BlockSpec((BE * CO // 128, 128), lambda i: (i, 0)),
      out_shape=jax.ShapeDtypeStruct((EP * CO // 128, 128), jnp.float32),
  )


def _dinv_body(d0_ref, d1_ref, out_ref):
  deg = d0_ref[...] + d1_ref[...]
  out_ref[...] = 1.0 / jnp.maximum(deg, 1.0)


PBN = BN * CO // 128
PNA = NA * CO // 128

_dinv_k = pl.pallas_call(
    _dinv_body,
    grid=(NA // BN,),
    in_specs=[
        pl.BlockSpec((PBN, 128), lambda i: (i + 2 * (NA // BN), 0)),
        pl.BlockSpec((PBN, 128), lambda i: (i + 3 * (NA // BN), 0)),
    ],
    out_specs=pl.BlockSpec((PBN, 128), lambda i: (i, 0)),
    out_shape=jax.ShapeDtypeStruct((PNA, 128), jnp.float32),
)


def _comb_body_n(nparts, refs):
  p0_ref, p1_ref, dinv_ref = refs[0], refs[1], refs[2]
  x_refs = refs[3:3 + nparts]
  rk_refs = refs[3 + nparts:3 + 2 * nparts]
  b_ref = refs[3 + 2 * nparts]
  out_ref = refs[4 + 2 * nparts]
  agg = (p0_ref[...] + p1_ref[...]) * dinv_ref[...]
  xr = b_ref[...]
  for xref, rkref in zip(x_refs, rk_refs):
    xr = xr + jnp.dot(xref[...], rkref[...],
                      preferred_element_type=jnp.float32)
  o = agg + xr
  out_ref[...] = jnp.minimum(jnp.maximum(o, 0.0), 6.0)


@functools.lru_cache(maxsize=None)
def _make_combine(nparts):
  x_specs = [pl.BlockSpec((PBN, 128), lambda i: (i, 0))
             for _ in range(nparts)]
  rk_specs = [pl.BlockSpec((128, 128), lambda i: (0, 0))
              for _ in range(nparts)]
  return pl.pallas_call(
      lambda *refs: _comb_body_n(nparts, refs),
      grid=(NA // BN,),
      in_specs=[
          pl.BlockSpec((PBN, 128), lambda i: (i, 0)),
          pl.BlockSpec((PBN, 128), lambda i: (i + NA // BN, 0)),
          pl.BlockSpec((PBN, 128), lambda i: (i, 0)),
          *x_specs,
          *rk_specs,
          pl.BlockSpec((1, 128), lambda i: (0, 0)),
      ],
      out_specs=pl.BlockSpec((PBN, 128), lambda i: (i, 0)),
      out_shape=jax.ShapeDtypeStruct((PNA, 128), jnp.float32),
  )


def _final_body(flat_ref, w_ref, b_ref, batch_ref, out_ref):
  m = jnp.max(batch_ref[...])
  delta = (m + 1 - BATCH).astype(jnp.float32)
  out_ref[...] = jnp.dot(flat_ref[...], w_ref[...],
                         preferred_element_type=jnp.float32) \
      + b_ref[...] + delta


_final_k = pl.pallas_call(
    _final_body,
    out_shape=jax.ShapeDtypeStruct((BATCH, LIN_OUT), jnp.float32),
)


def kernel(x, edge_index, edge_attr, batch, pos, params):
  del edge_attr
  f32 = jnp.float32
  src = edge_index[0]
  dst = edge_index[1]
  pad_e = EP - E
  src2 = jnp.concatenate(
      [src, jnp.zeros((pad_e,), jnp.int32)]).reshape(EP // 128, 128)
  dst2 = jnp.concatenate(
      [dst, jnp.full((pad_e,), N, jnp.int32)]).reshape(EP // 128, 128)
  posp = jnp.concatenate([pos, jnp.zeros((pad_e, 2), f32)], axis=0)
  zeros_n = jnp.zeros((NA, CO), f32)
  ones_sc = jnp.ones((128, CO), f32)

  xpad = jnp.zeros((NA, CO), f32).at[:N].set(x)
  xpad_p = xpad.reshape(PNA, 128)

  def wmat(l, ci):
    return params['conv%d_w' % l].transpose(2, 0, 1).reshape(CO, S * ci)

  post = posp.reshape(EP // BE, PPE, 8, 2).transpose(0, 2, 1, 3) \
      .reshape(EP, 2).T

  eye8 = jnp.eye(8, dtype=f32)

  def kron8(m):
    return jnp.kron(eye8, m)

  def rk(l, nparts):
    root = params['conv%d_root' % l]
    return [kron8(root[k * CO:(k + 1) * CO]) for k in range(nparts)]

  def bt(l):
    return jnp.tile(params['conv%d_b' % l].reshape(1, CO), (1, 8))

  def gather_p(hs):
    lins = [h.reshape(NA, CO) for h in hs]
    outs = _make_gather(len(hs))(*lins, src2)
    if not isinstance(outs, (list, tuple)):
      outs = [outs]
    return [o.reshape(EP * CO // 128, 128) for o in outs]

  def spmm(hs, l):
    xjs = gather_p(hs)
    msg = _make_msg(CO * len(hs))(post, *xjs, wmat(l, CO * len(hs)))
    p = _scatter_k(msg.reshape(EP, CO), dst2, zeros_n)
    return p.reshape(2 * PNA, 128)

  xjs = gather_p([xpad_p])
  msg = _make_msg(CO)(post, xjs[0], wmat(0, CO))
  p = _scatter_deg_k(msg.reshape(EP, CO), dst2, zeros_n, ones_sc)
  p = p.reshape(4 * PNA, 128)
  dinv = _dinv_k(p, p)
  o1 = _make_combine(1)(p, p, dinv, xpad_p, *rk(0, 1), bt(0))
  p = spmm([o1], 1)
  o2 = _make_combine(1)(p, p, dinv, o1, *rk(1, 1), bt(1))
  p = spmm([o2], 2)
  o3 = _make_combine(1)(p, p, dinv, o2, *rk(2, 1), bt(2))
  p = spmm([o3, o2], 3)
  d3 = _make_combine(2)(p, p, dinv, o3, o2, *rk(3, 2), bt(3))
  p = spmm([d3, o1], 4)
  d = _make_combine(2)(p, p, dinv, d3, o1, *rk(4, 2), bt(4))

  flat = jax.lax.slice(d, (0, 0), (N * CO // 128, 128)).reshape(BATCH, LIN_IN)
  batch2 = batch.reshape(BATCH, N // BATCH)
  return _final_k(flat, params['lin_w'],
                  params['lin_b'].reshape(1, LIN_OUT), batch2)
